# baseline (speedup 1.0000x reference)
import functools

import jax
import jax.numpy as jnp
from jax.experimental import pallas as pl

T = 50000
NN = 8
H = 128
L = 56
V = 780
B = 1024

TT = 625
NTILES = T // TT


def _fused_body(cur_x_ref, h_ref, o_ref, bidx_ref, ptgt_ref, stgt_ref,
                emb_ref, tvs_ref,
                wz1_ref, wz2_ref, wzb_ref,
                wr_ref, wrb_ref, ur_ref,
                wh1_ref, wh2_ref, whb_ref,
                w1_ref, w2_ref, wb_ref,
                u1_ref, u2_ref, u3_ref, ub_ref,
                wo_ref, wob_ref, us_ref, usb_ref,
                pl_out, sl_out, pa_out, sa_out):
    i = pl.program_id(0)

    ids = cur_x_ref[0]
    bidx = bidx_ref[0]
    ptgt = ptgt_ref[0]
    st = stgt_ref[0].astype(jnp.float32)

    h = h_ref[...]
    o = o_ref[...]

    viota = jax.lax.broadcasted_iota(jnp.int32, (TT, V), 1)
    onehot_x = (viota == ids).astype(jnp.float32)
    x = jnp.dot(onehot_x, emb_ref[...], preferred_element_type=jnp.float32)

    biota = jax.lax.broadcasted_iota(jnp.int32, (TT, B), 1)
    onehot_b = (biota == bidx).astype(jnp.float32)
    tv = jnp.dot(onehot_b, tvs_ref[...], preferred_element_type=jnp.float32)

    sum_h = jnp.sum(h, axis=1)
    cur_o = jnp.sum(o, axis=1)
    z = jax.nn.sigmoid(
        jnp.dot(x, wz1_ref[...], preferred_element_type=jnp.float32)
        + jnp.dot(sum_h, wz2_ref[...], preferred_element_type=jnp.float32)
        + wzb_ref[...])
    r1 = jnp.dot(x, wr_ref[...], preferred_element_type=jnp.float32) + wrb_ref[...]
    hm = h.reshape(TT * NN, H)
    r2 = jnp.dot(hm, ur_ref[...], preferred_element_type=jnp.float32)
    r = jax.nn.sigmoid(r1[:, None, :] + r2.reshape(TT, NN, H))
    sum_gated = jnp.sum(r * h, axis=1)
    pre_h = jnp.tanh(
        jnp.dot(x, wh1_ref[...], preferred_element_type=jnp.float32)
        + jnp.dot(sum_gated, wh2_ref[...], preferred_element_type=jnp.float32)
        + whb_ref[...])
    new_h = (1.0 - z) * sum_h + z * pre_h

    pv = jax.nn.relu(
        jnp.dot(new_h, w1_ref[...], preferred_element_type=jnp.float32)
        + jnp.dot(tv, w2_ref[...], preferred_element_type=jnp.float32)
        + wb_ref[...])
    ps = jnp.dot(pv, wo_ref[...], preferred_element_type=jnp.float32) + wob_ref[...]
    m = jnp.max(ps, axis=1, keepdims=True)
    lse = m + jnp.log(jnp.sum(jnp.exp(ps - m), axis=1, keepdims=True))
    tmask = viota == ptgt
    tgt_logit = jnp.sum(jnp.where(tmask, ps, 0.0), axis=1, keepdims=True)
    pl_sum = jnp.sum(lse - tgt_logit, axis=0, keepdims=True)

    pa_sum = jnp.sum((tgt_logit == m).astype(jnp.float32), axis=0, keepdims=True)

    sv = jax.nn.relu(
        jnp.dot(x, u1_ref[...], preferred_element_type=jnp.float32)
        + jnp.dot(cur_o, u2_ref[...], preferred_element_type=jnp.float32)
        + jnp.dot(tv, u3_ref[...], preferred_element_type=jnp.float32)
        + ub_ref[...])
    ss = jnp.sum(sv * us_ref[...], axis=1, keepdims=True) + usb_ref[...]
    sp = jnp.maximum(ss, 0.0) + jnp.log1p(jnp.exp(-jnp.abs(ss)))
    sl_sum = jnp.sum(sp - ss * st, axis=0, keepdims=True)
    stops = (ss >= 0.0).astype(jnp.float32)
    sa_sum = jnp.sum((stops == st).astype(jnp.float32), axis=0, keepdims=True)

    @pl.when(i == 0)
    def _():
        pl_out[...] = jnp.zeros_like(pl_out)
        sl_out[...] = jnp.zeros_like(sl_out)
        pa_out[...] = jnp.zeros_like(pa_out)
        sa_out[...] = jnp.zeros_like(sa_out)

    pl_out[...] += pl_sum
    sl_out[...] += sl_sum
    pa_out[...] += pa_sum
    sa_out[...] += sa_sum


def kernel(cur_x, h_nei, o_nei, batch_idx, tree_vecs, pred_targets, stop_targets,
           embedding, Wz_w, Wz_b, Wr_w, Wr_b, Ur_w, Wh_w, Wh_b,
           W_w, W_b, U_w, U_b, Wo_w, Wo_b, Us_w, Us_b):
    idx3 = lambda a: a.reshape(NTILES, TT, 1)
    row = lambda b: b.reshape(1, -1)

    tile_spec = lambda blk: pl.BlockSpec(blk, lambda i: (i, 0, 0))
    nei_spec = pl.BlockSpec((TT, NN, H), lambda i: (i, 0, 0))
    rep2 = lambda shape: pl.BlockSpec(shape, lambda i: (0, 0))

    args = (
        idx3(cur_x), h_nei, o_nei,
        idx3(batch_idx), idx3(pred_targets), idx3(stop_targets),
        embedding, tree_vecs,
        Wz_w[:H], Wz_w[H:], row(Wz_b),
        Wr_w, row(Wr_b), Ur_w,
        Wh_w[:H], Wh_w[H:], row(Wh_b),
        W_w[:H], W_w[H:], row(W_b),
        U_w[:H], U_w[H:2 * H], U_w[2 * H:], row(U_b),
        Wo_w, row(Wo_b), Us_w.reshape(1, H), Us_b.reshape(1, 1),
    )
    in_specs = [
        tile_spec((1, TT, 1)), nei_spec, nei_spec,
        tile_spec((1, TT, 1)), tile_spec((1, TT, 1)), tile_spec((1, TT, 1)),
        rep2((V, H)), rep2((B, L)),
        rep2((H, H)), rep2((H, H)), rep2((1, H)),
        rep2((H, H)), rep2((1, H)), rep2((H, H)),
        rep2((H, H)), rep2((H, H)), rep2((1, H)),
        rep2((H, H)), rep2((L, H)), rep2((1, H)),
        rep2((H, H)), rep2((H, H)), rep2((L, H)), rep2((1, H)),
        rep2((H, V)), rep2((1, V)), rep2((1, H)), rep2((1, 1)),
    ]
    out_specs = [pl.BlockSpec((1, 1), lambda i: (0, 0))] * 4
    out_shape = [jax.ShapeDtypeStruct((1, 1), jnp.float32)] * 4

    pls, sls, pas, sas = pl.pallas_call(
        _fused_body,
        grid=(NTILES,),
        in_specs=in_specs,
        out_specs=out_specs,
        out_shape=out_shape,
    )(*args)

    nB = jnp.float32(B)
    nT = jnp.float32(T)
    return (pls[0, 0] / nB, sls[0, 0] / nB, pas[0, 0] / nT, sas[0, 0] / nT)


# speedup vs baseline: 1.1850x; 1.1850x over previous
import functools

import jax
import jax.numpy as jnp
from jax import lax
from jax.experimental import pallas as pl
from jax.experimental.pallas import tpu as pltpu
from jax.experimental.pallas import tpu_sc as plsc

T = 50000
NN = 8
H = 128
L = 56
V = 780
B = 1024

TT = 1000
NTILES = T // TT

LP = 128
NC = 2
NS = 16
NW = NC * NS
CHUNK = 392
NCHUNK = 4
PER_W = CHUNK * NCHUNK
T_PAD = NW * PER_W


def _gather_body(curx_hbm, bidx_hbm, emb_hbm, tvs_hbm, x_hbm, tv_hbm,
                 idxx_v, idxb_v, xrows_v, tvrows_v, semx, semb):
    wid = lax.axis_index("s") * NC + lax.axis_index("c")
    base0 = wid * PER_W
    for c in range(NCHUNK):
        base = base0 + c * CHUNK
        pltpu.sync_copy(curx_hbm.at[pl.ds(base, CHUNK)], idxx_v)
        pltpu.sync_copy(bidx_hbm.at[pl.ds(base, CHUNK)], idxb_v)
        cpx = pltpu.async_copy(emb_hbm.at[idxx_v], xrows_v, semx)
        cpb = pltpu.async_copy(tvs_hbm.at[idxb_v], tvrows_v, semb)
        cpx.wait()
        cpb.wait()
        pltpu.sync_copy(xrows_v, x_hbm.at[pl.ds(base, CHUNK)])
        pltpu.sync_copy(tvrows_v, tv_hbm.at[pl.ds(base, CHUNK)])


def _sc_gather(cur_x_pad, batch_idx_pad, embedding, tree_vecs_pad):
    mesh = plsc.VectorSubcoreMesh(core_axis_name="c", subcore_axis_name="s")
    return pl.kernel(
        _gather_body,
        out_type=[
            jax.ShapeDtypeStruct((T_PAD, H), jnp.float32),
            jax.ShapeDtypeStruct((T_PAD, LP), jnp.float32),
        ],
        mesh=mesh,
        scratch_types=[
            pltpu.VMEM((CHUNK,), jnp.int32),
            pltpu.VMEM((CHUNK,), jnp.int32),
            pltpu.VMEM((CHUNK, H), jnp.float32),
            pltpu.VMEM((CHUNK, LP), jnp.float32),
            pltpu.SemaphoreType.DMA,
            pltpu.SemaphoreType.DMA,
        ],
    )(cur_x_pad, batch_idx_pad, embedding, tree_vecs_pad)


def _fused_body(x_ref, h_ref, o_ref, tv_ref, ptgt_ref, stgt_ref,
                wz1_ref, wz2_ref, wzb_ref,
                wr_ref, wrb_ref, ur_ref,
                wh1_ref, wh2_ref, whb_ref,
                w1_ref, w2_ref, wb_ref,
                u1_ref, u2_ref, u3_ref, ub_ref,
                wo_ref, wob_ref, us_ref, usb_ref,
                pl_out, sl_out, pa_out, sa_out):
    i = pl.program_id(0)

    ptgt = ptgt_ref[0]
    st = stgt_ref[0].astype(jnp.float32)

    x = x_ref[...]
    tv = tv_ref[...]
    h = h_ref[...]
    o = o_ref[...]

    sum_h = jnp.sum(h, axis=1)
    cur_o = jnp.sum(o, axis=1)
    z = jax.nn.sigmoid(
        jnp.dot(x, wz1_ref[...], preferred_element_type=jnp.float32)
        + jnp.dot(sum_h, wz2_ref[...], preferred_element_type=jnp.float32)
        + wzb_ref[...])
    r1 = jnp.dot(x, wr_ref[...], preferred_element_type=jnp.float32) + wrb_ref[...]
    hm = h.reshape(TT * NN, H)
    r2 = jnp.dot(hm, ur_ref[...], preferred_element_type=jnp.float32)
    r = jax.nn.sigmoid(r1[:, None, :] + r2.reshape(TT, NN, H))
    sum_gated = jnp.sum(r * h, axis=1)
    pre_h = jnp.tanh(
        jnp.dot(x, wh1_ref[...], preferred_element_type=jnp.float32)
        + jnp.dot(sum_gated, wh2_ref[...], preferred_element_type=jnp.float32)
        + whb_ref[...])
    new_h = (1.0 - z) * sum_h + z * pre_h

    pv = jax.nn.relu(
        jnp.dot(new_h, w1_ref[...], preferred_element_type=jnp.float32)
        + jnp.dot(tv, w2_ref[...], preferred_element_type=jnp.float32)
        + wb_ref[...])
    ps = jnp.dot(pv, wo_ref[...], preferred_element_type=jnp.float32) + wob_ref[...]
    m = jnp.max(ps, axis=1, keepdims=True)
    lse = m + jnp.log(jnp.sum(jnp.exp(ps - m), axis=1, keepdims=True))
    viota = jax.lax.broadcasted_iota(jnp.int32, (TT, V), 1)
    tmask = viota == ptgt
    tgt_logit = jnp.sum(jnp.where(tmask, ps, 0.0), axis=1, keepdims=True)
    pl_sum = jnp.sum(lse - tgt_logit, axis=0, keepdims=True)

    pa_sum = jnp.sum((tgt_logit == m).astype(jnp.float32), axis=0, keepdims=True)

    sv = jax.nn.relu(
        jnp.dot(x, u1_ref[...], preferred_element_type=jnp.float32)
        + jnp.dot(cur_o, u2_ref[...], preferred_element_type=jnp.float32)
        + jnp.dot(tv, u3_ref[...], preferred_element_type=jnp.float32)
        + ub_ref[...])
    ss = jnp.sum(sv * us_ref[...], axis=1, keepdims=True) + usb_ref[...]
    sp = jnp.maximum(ss, 0.0) + jnp.log1p(jnp.exp(-jnp.abs(ss)))
    sl_sum = jnp.sum(sp - ss * st, axis=0, keepdims=True)
    stops = (ss >= 0.0).astype(jnp.float32)
    sa_sum = jnp.sum((stops == st).astype(jnp.float32), axis=0, keepdims=True)

    @pl.when(i == 0)
    def _():
        pl_out[...] = jnp.zeros_like(pl_out)
        sl_out[...] = jnp.zeros_like(sl_out)
        pa_out[...] = jnp.zeros_like(pa_out)
        sa_out[...] = jnp.zeros_like(sa_out)

    pl_out[...] += pl_sum
    sl_out[...] += sl_sum
    pa_out[...] += pa_sum
    sa_out[...] += sa_sum


def kernel(cur_x, h_nei, o_nei, batch_idx, tree_vecs, pred_targets, stop_targets,
           embedding, Wz_w, Wz_b, Wr_w, Wr_b, Ur_w, Wh_w, Wh_b,
           W_w, W_b, U_w, U_b, Wo_w, Wo_b, Us_w, Us_b):
    cur_x_pad = jnp.pad(cur_x, (0, T_PAD - T))
    batch_idx_pad = jnp.pad(batch_idx, (0, T_PAD - T))
    tree_vecs_pad = jnp.pad(tree_vecs, ((0, 0), (0, LP - L)))
    x_all, tv_all = _sc_gather(cur_x_pad, batch_idx_pad, embedding,
                               tree_vecs_pad)

    idx3 = lambda a: a.reshape(NTILES, TT, 1)
    row = lambda b: b.reshape(1, -1)
    padw = lambda w: jnp.pad(w, ((0, LP - L), (0, 0)))

    tile_spec = lambda blk: pl.BlockSpec(blk, lambda i: (i, 0, 0))
    nei_spec = pl.BlockSpec((TT, NN, H), lambda i: (i, 0, 0))
    row_spec = lambda n: pl.BlockSpec((TT, n), lambda i: (i, 0))
    rep2 = lambda shape: pl.BlockSpec(shape, lambda i: (0, 0))

    args = (
        x_all, h_nei, o_nei, tv_all,
        idx3(pred_targets), idx3(stop_targets),
        Wz_w[:H], Wz_w[H:], row(Wz_b),
        Wr_w, row(Wr_b), Ur_w,
        Wh_w[:H], Wh_w[H:], row(Wh_b),
        W_w[:H], padw(W_w[H:]), row(W_b),
        U_w[:H], U_w[H:2 * H], padw(U_w[2 * H:]), row(U_b),
        Wo_w, row(Wo_b), Us_w.reshape(1, H), Us_b.reshape(1, 1),
    )
    in_specs = [
        row_spec(H), nei_spec, nei_spec, row_spec(LP),
        tile_spec((1, TT, 1)), tile_spec((1, TT, 1)),
        rep2((H, H)), rep2((H, H)), rep2((1, H)),
        rep2((H, H)), rep2((1, H)), rep2((H, H)),
        rep2((H, H)), rep2((H, H)), rep2((1, H)),
        rep2((H, H)), rep2((LP, H)), rep2((1, H)),
        rep2((H, H)), rep2((H, H)), rep2((LP, H)), rep2((1, H)),
        rep2((H, V)), rep2((1, V)), rep2((1, H)), rep2((1, 1)),
    ]
    out_specs = [pl.BlockSpec((1, 1), lambda i: (0, 0))] * 4
    out_shape = [jax.ShapeDtypeStruct((1, 1), jnp.float32)] * 4

    pls, sls, pas, sas = pl.pallas_call(
        _fused_body,
        grid=(NTILES,),
        in_specs=in_specs,
        out_specs=out_specs,
        out_shape=out_shape,
    )(*args)

    nB = jnp.float32(B)
    nT = jnp.float32(T)
    return (pls[0, 0] / nB, sls[0, 0] / nB, pas[0, 0] / nT, sas[0, 0] / nT)


# speedup vs baseline: 1.2599x; 1.0632x over previous
import functools

import jax
import jax.numpy as jnp
from jax import lax
from jax.experimental import pallas as pl
from jax.experimental.pallas import tpu as pltpu
from jax.experimental.pallas import tpu_sc as plsc

T = 50000
NN = 8
H = 128
L = 56
V = 780
B = 1024

TT = 2000
NTILES = T // TT

LP = 128
NC = 2
NS = 16
NW = NC * NS
CHUNK = 392
NCHUNK = 4
PER_W = CHUNK * NCHUNK
T_PAD = NW * PER_W


def _gather_body(curx_hbm, bidx_hbm, emb_hbm, tvs_hbm, x_hbm, tv_hbm,
                 idxx_v, idxb_v, xrows_v, tvrows_v, semx, semb):
    wid = lax.axis_index("s") * NC + lax.axis_index("c")
    base0 = wid * PER_W
    for c in range(NCHUNK):
        base = base0 + c * CHUNK
        pltpu.sync_copy(curx_hbm.at[pl.ds(base, CHUNK)], idxx_v)
        pltpu.sync_copy(bidx_hbm.at[pl.ds(base, CHUNK)], idxb_v)
        cpx = pltpu.async_copy(emb_hbm.at[idxx_v], xrows_v, semx)
        cpb = pltpu.async_copy(tvs_hbm.at[idxb_v], tvrows_v, semb)
        cpx.wait()
        cpb.wait()
        pltpu.sync_copy(xrows_v, x_hbm.at[pl.ds(base, CHUNK)])
        pltpu.sync_copy(tvrows_v, tv_hbm.at[pl.ds(base, CHUNK)])


def _sc_gather(cur_x_pad, batch_idx_pad, embedding, tree_vecs_pad):
    mesh = plsc.VectorSubcoreMesh(core_axis_name="c", subcore_axis_name="s")
    return pl.kernel(
        _gather_body,
        out_type=[
            jax.ShapeDtypeStruct((T_PAD, H), jnp.float32),
            jax.ShapeDtypeStruct((T_PAD, LP), jnp.float32),
        ],
        mesh=mesh,
        scratch_types=[
            pltpu.VMEM((CHUNK,), jnp.int32),
            pltpu.VMEM((CHUNK,), jnp.int32),
            pltpu.VMEM((CHUNK, H), jnp.float32),
            pltpu.VMEM((CHUNK, LP), jnp.float32),
            pltpu.SemaphoreType.DMA,
            pltpu.SemaphoreType.DMA,
        ],
    )(cur_x_pad, batch_idx_pad, embedding, tree_vecs_pad)


def _fused_body(x_ref, h_ref, o_ref, tv_ref, ptgt_ref, stgt_ref,
                wz1_ref, wz2_ref, wzb_ref,
                wr_ref, wrb_ref, ur_ref,
                wh1_ref, wh2_ref, whb_ref,
                w1_ref, w2_ref, wb_ref,
                u1_ref, u2_ref, u3_ref, ub_ref,
                wo_ref, wob_ref, us_ref, usb_ref,
                pl_out, sl_out, pa_out, sa_out):
    i = pl.program_id(0)

    ptgt = ptgt_ref[0]
    st = stgt_ref[0].astype(jnp.float32)

    x = x_ref[...]
    tv = tv_ref[...]
    h = h_ref[...]
    o = o_ref[...]

    sum_h = jnp.sum(h, axis=1)
    cur_o = jnp.sum(o, axis=1)
    z = jax.nn.sigmoid(
        jnp.dot(x, wz1_ref[...], preferred_element_type=jnp.float32)
        + jnp.dot(sum_h, wz2_ref[...], preferred_element_type=jnp.float32)
        + wzb_ref[...])
    r1 = jnp.dot(x, wr_ref[...], preferred_element_type=jnp.float32) + wrb_ref[...]
    hm = h.reshape(TT * NN, H)
    r2 = jnp.dot(hm, ur_ref[...], preferred_element_type=jnp.float32)
    r = jax.nn.sigmoid(r1[:, None, :] + r2.reshape(TT, NN, H))
    sum_gated = jnp.sum(r * h, axis=1)
    pre_h = jnp.tanh(
        jnp.dot(x, wh1_ref[...], preferred_element_type=jnp.float32)
        + jnp.dot(sum_gated, wh2_ref[...], preferred_element_type=jnp.float32)
        + whb_ref[...])
    new_h = (1.0 - z) * sum_h + z * pre_h

    pv = jax.nn.relu(
        jnp.dot(new_h, w1_ref[...], preferred_element_type=jnp.float32)
        + jnp.dot(tv, w2_ref[...], preferred_element_type=jnp.float32)
        + wb_ref[...])
    ps = jnp.dot(pv, wo_ref[...], preferred_element_type=jnp.float32) + wob_ref[...]
    m = jnp.max(ps, axis=1, keepdims=True)
    lse = m + jnp.log(jnp.sum(jnp.exp(ps - m), axis=1, keepdims=True))
    viota = jax.lax.broadcasted_iota(jnp.int32, (TT, V), 1)
    tmask = viota == ptgt
    tgt_logit = jnp.sum(jnp.where(tmask, ps, 0.0), axis=1, keepdims=True)
    pl_sum = jnp.sum(lse - tgt_logit, axis=0, keepdims=True)

    pa_sum = jnp.sum((tgt_logit == m).astype(jnp.float32), axis=0, keepdims=True)

    sv = jax.nn.relu(
        jnp.dot(x, u1_ref[...], preferred_element_type=jnp.float32)
        + jnp.dot(cur_o, u2_ref[...], preferred_element_type=jnp.float32)
        + jnp.dot(tv, u3_ref[...], preferred_element_type=jnp.float32)
        + ub_ref[...])
    ss = jnp.sum(sv * us_ref[...], axis=1, keepdims=True) + usb_ref[...]
    sp = jnp.maximum(ss, 0.0) + jnp.log1p(jnp.exp(-jnp.abs(ss)))
    sl_sum = jnp.sum(sp - ss * st, axis=0, keepdims=True)
    stops = (ss >= 0.0).astype(jnp.float32)
    sa_sum = jnp.sum((stops == st).astype(jnp.float32), axis=0, keepdims=True)

    @pl.when(i == 0)
    def _():
        pl_out[...] = jnp.zeros_like(pl_out)
        sl_out[...] = jnp.zeros_like(sl_out)
        pa_out[...] = jnp.zeros_like(pa_out)
        sa_out[...] = jnp.zeros_like(sa_out)

    pl_out[...] += pl_sum
    sl_out[...] += sl_sum
    pa_out[...] += pa_sum
    sa_out[...] += sa_sum


def kernel(cur_x, h_nei, o_nei, batch_idx, tree_vecs, pred_targets, stop_targets,
           embedding, Wz_w, Wz_b, Wr_w, Wr_b, Ur_w, Wh_w, Wh_b,
           W_w, W_b, U_w, U_b, Wo_w, Wo_b, Us_w, Us_b):
    cur_x_pad = jnp.pad(cur_x, (0, T_PAD - T))
    batch_idx_pad = jnp.pad(batch_idx, (0, T_PAD - T))
    tree_vecs_pad = jnp.pad(tree_vecs, ((0, 0), (0, LP - L)))
    x_all, tv_all = _sc_gather(cur_x_pad, batch_idx_pad, embedding,
                               tree_vecs_pad)

    idx3 = lambda a: a.reshape(NTILES, TT, 1)
    row = lambda b: b.reshape(1, -1)
    padw = lambda w: jnp.pad(w, ((0, LP - L), (0, 0)))

    tile_spec = lambda blk: pl.BlockSpec(blk, lambda i: (i, 0, 0))
    nei_spec = pl.BlockSpec((TT, NN, H), lambda i: (i, 0, 0))
    row_spec = lambda n: pl.BlockSpec((TT, n), lambda i: (i, 0))
    rep2 = lambda shape: pl.BlockSpec(shape, lambda i: (0, 0))

    args = (
        x_all, h_nei, o_nei, tv_all,
        idx3(pred_targets), idx3(stop_targets),
        Wz_w[:H], Wz_w[H:], row(Wz_b),
        Wr_w, row(Wr_b), Ur_w,
        Wh_w[:H], Wh_w[H:], row(Wh_b),
        W_w[:H], padw(W_w[H:]), row(W_b),
        U_w[:H], U_w[H:2 * H], padw(U_w[2 * H:]), row(U_b),
        Wo_w, row(Wo_b), Us_w.reshape(1, H), Us_b.reshape(1, 1),
    )
    in_specs = [
        row_spec(H), nei_spec, nei_spec, row_spec(LP),
        tile_spec((1, TT, 1)), tile_spec((1, TT, 1)),
        rep2((H, H)), rep2((H, H)), rep2((1, H)),
        rep2((H, H)), rep2((1, H)), rep2((H, H)),
        rep2((H, H)), rep2((H, H)), rep2((1, H)),
        rep2((H, H)), rep2((LP, H)), rep2((1, H)),
        rep2((H, H)), rep2((H, H)), rep2((LP, H)), rep2((1, H)),
        rep2((H, V)), rep2((1, V)), rep2((1, H)), rep2((1, 1)),
    ]
    out_specs = [pl.BlockSpec((1, 1), lambda i: (0, 0))] * 4
    out_shape = [jax.ShapeDtypeStruct((1, 1), jnp.float32)] * 4

    pls, sls, pas, sas = pl.pallas_call(
        _fused_body,
        grid=(NTILES,),
        in_specs=in_specs,
        out_specs=out_specs,
        out_shape=out_shape,
    )(*args)

    nB = jnp.float32(B)
    nT = jnp.float32(T)
    return (pls[0, 0] / nB, sls[0, 0] / nB, pas[0, 0] / nT, sas[0, 0] / nT)
